# baseline (speedup 1.0000x reference)
import functools
import jax
import jax.numpy as jnp
from jax import lax
from jax.experimental import pallas as pl
from jax.experimental.pallas import tpu as pltpu
from jax.experimental.pallas import tpu_sc as plsc

N = 10000
E = 160000
H = 128
NH = 8
HD = 16
NRBF = 50
NRBF_P = 64
L = 2
CUTOFF = 5.0

NC = 2
NS = 16
NW = NC * NS
CH = 128
EP = 163840
NP = 12288
NACC = 10240
ROWS_S = NACC // NS


def _silu(x):
    return x * jax.nn.sigmoid(x)


def _ln_block(x, g, b):
    m = jnp.mean(x, axis=-1, keepdims=True)
    v = jnp.mean((x - m) ** 2, axis=-1, keepdims=True)
    return (x - m) / jnp.sqrt(v + 1e-5) * g + b


def _sc_gather(table, idx, D):
    EPi = idx.shape[0]
    per_w = EPi // NW
    dt = table.dtype
    nbytes = D * dt.itemsize
    GB = 64 if nbytes >= 2048 else CH
    nch = per_w // GB
    mesh = plsc.VectorSubcoreMesh(core_axis_name="c", subcore_axis_name="s")

    @functools.partial(
        pl.kernel,
        mesh=mesh,
        out_type=jax.ShapeDtypeStruct((EPi, D), dt),
        scratch_types=[
            pltpu.VMEM((per_w,), jnp.int32),
            pltpu.VMEM((2, GB, D), dt),
            pltpu.SemaphoreType.DMA,
            pltpu.SemaphoreType.DMA,
            pltpu.SemaphoreType.DMA,
            pltpu.SemaphoreType.DMA,
        ],
    )
    def k(table_hbm, idx_hbm, out_hbm, idx_v, rows_v, g0, g1, s0, s1):
        gsem = (g0, g1)
        ssem = (s0, s1)
        wid = lax.axis_index("s") * NC + lax.axis_index("c")
        base = wid * per_w
        pltpu.sync_copy(idx_hbm.at[pl.ds(base, per_w)], idx_v)

        def gather_cp(j, b):
            return pltpu.make_async_copy(table_hbm.at[idx_v.at[pl.ds(j * GB, GB)]],
                                         rows_v.at[b], gsem[b])

        def store_cp(j, b):
            return pltpu.make_async_copy(
                rows_v.at[b], out_hbm.at[pl.ds(base + j * GB, GB)], ssem[b])

        def step(j, b):
            nxt = j + 1

            @pl.when(nxt < nch)
            def _():
                gather_cp(nxt, 1 - b).start()

            gather_cp(j, b).wait()
            cp = store_cp(j, b)
            cp.start()
            cp.wait()

        gather_cp(0, 0).start()

        def body(jo, carry):
            step(2 * jo, 0)
            step(2 * jo + 1, 1)
            return carry

        lax.fori_loop(0, nch // 2, body, 0)
        if nch % 2:
            step(nch - 1, (nch - 1) % 2)

    return k(table, idx)


def _sc_scatter(msgs, idx, zeros, C):
    EPi = idx.shape[0]
    per_w = EPi // NW
    nch = per_w // CH
    idx2 = idx.reshape(-1, CH)
    mesh = plsc.VectorSubcoreMesh(core_axis_name="c", subcore_axis_name="s")

    @functools.partial(
        pl.kernel,
        mesh=mesh,
        out_type=jax.ShapeDtypeStruct((NC, C, NACC, H), jnp.float32),
        scratch_types=[
            pltpu.VMEM((nch, CH), jnp.int32),
            pltpu.VMEM((2, CH, H), jnp.float32),
            pltpu.VMEM_SHARED((NACC, H), jnp.float32),
            pltpu.SemaphoreType.DMA,
            pltpu.SemaphoreType.DMA,
        ],
    )
    def k(msgs_hbm, idx_hbm, zeros_hbm, out_hbm, idx_v, msg_v, acc, l0, l1):
        lsem = (l0, l1)
        cid = lax.axis_index("c")
        sid = lax.axis_index("s")
        wid = sid * NC + cid
        base = wid * per_w
        pltpu.sync_copy(idx_hbm.at[pl.ds(wid * nch, nch)], idx_v)
        for c in range(C):
            pltpu.sync_copy(zeros_hbm.at[pl.ds(sid * ROWS_S, ROWS_S)],
                            acc.at[pl.ds(sid * ROWS_S, ROWS_S)])
            plsc.subcore_barrier()

            def load_cp(j, b):
                return pltpu.make_async_copy(
                    msgs_hbm.at[pl.ds(base + j * CH, CH), pl.ds(c * H, H)],
                    msg_v.at[b], lsem[b])

            def step(j, b):
                nxt = j + 1

                @pl.when(nxt < nch)
                def _():
                    load_cp(nxt, 1 - b).start()

                load_cp(j, b).wait()
                pltpu.sync_copy(msg_v.at[b], acc.at[idx_v.at[j]], add=True)

            load_cp(0, 0).start()

            def body(jo, carry):
                step(2 * jo, 0)
                step(2 * jo + 1, 1)
                return carry

            lax.fori_loop(0, nch // 2, body, 0)
            if nch % 2:
                step(nch - 1, (nch - 1) % 2)
            plsc.subcore_barrier()
            pltpu.sync_copy(acc.at[pl.ds(sid * ROWS_S, ROWS_S)],
                            out_hbm.at[cid, c, pl.ds(sid * ROWS_S, ROWS_S)])
            plsc.subcore_barrier()

    return k(msgs, idx2, zeros)


def _geo_kernel(ps_ref, pd_ref, means_ref, betas_ref, out_ref):
    B = ps_ref.shape[0]
    ps = ps_ref[...]
    pd = pd_ref[...]
    pv = pd[:, 0:3] - ps[:, 0:3]
    d2 = jnp.sum(pv * pv, axis=1, keepdims=True)
    w = jnp.sqrt(jnp.maximum(d2, 1e-12))
    attr = jnp.exp(-betas_ref[...] * (jnp.exp(-w) - means_ref[...]) ** 2)
    cc = 0.5 * (jnp.cos(w * (jnp.pi / CUTOFF)) + 1.0) * (w < CUTOFF).astype(jnp.float32)
    d_ij = pv / w
    ids = pl.program_id(0) * B + lax.broadcasted_iota(jnp.int32, (B, 1), 0)
    padf = (ids < E).astype(jnp.float32)
    mask = (ps[:, 3:4] != pd[:, 3:4]).astype(jnp.float32)
    out_ref[...] = jnp.concatenate(
        [attr, cc * mask * padf, cc * padf, d_ij,
         jnp.zeros((B, H - NRBF_P - 5), jnp.float32)], axis=1)


def _geo_edge(ps, pd, means, betas):
    B = 2048
    full = lambda s: pl.BlockSpec(s, lambda i: tuple(0 for _ in s))
    return pl.pallas_call(
        _geo_kernel,
        grid=(EP // B,),
        in_specs=[
            pl.BlockSpec((B, H), lambda i: (i, 0)),
            pl.BlockSpec((B, H), lambda i: (i, 0)),
            full((NRBF_P,)), full((NRBF_P,)),
        ],
        out_specs=pl.BlockSpec((B, H), lambda i: (i, 0)),
        out_shape=jax.ShapeDtypeStruct((EP, H), jnp.float32),
    )(ps, pd, means, betas)


def _ne_edge_kernel(geo_ref, xz_ref, W_ref, b_ref, out_ref):
    geo = geo_ref[...]
    attr = geo[:, 0:NRBF_P]
    wmsg = jnp.dot(attr, W_ref[...], preferred_element_type=jnp.float32) + b_ref[...]
    out_ref[...] = xz_ref[...] * wmsg * geo[:, NRBF_P:NRBF_P + 1]


def _ne_edge(geo, xz_src, W, b):
    B = 2048
    full = lambda s: pl.BlockSpec(s, lambda i: tuple(0 for _ in s))
    return pl.pallas_call(
        _ne_edge_kernel,
        grid=(EP // B,),
        in_specs=[
            pl.BlockSpec((B, H), lambda i: (i, 0)),
            pl.BlockSpec((B, H), lambda i: (i, 0)),
            full((NRBF_P, H)), full((H,)),
        ],
        out_specs=pl.BlockSpec((B, H), lambda i: (i, 0)),
        out_shape=jax.ShapeDtypeStruct((EP, H), jnp.float32),
    )(geo, xz_src, W, b)


def _ne_comb_kernel(xe_ref, sc_ref, W1_ref, W2_ref, b_ref, out_ref):
    agg = sc_ref[0, 0] + sc_ref[1, 0]
    out_ref[...] = (jnp.dot(xe_ref[...], W1_ref[...], preferred_element_type=jnp.float32)
                    + jnp.dot(agg, W2_ref[...], preferred_element_type=jnp.float32)
                    + b_ref[...])


def _ne_comb(x_emb, scat, W1, W2, b):
    B = 1000
    full = lambda s: pl.BlockSpec(s, lambda i: tuple(0 for _ in s))
    return pl.pallas_call(
        _ne_comb_kernel,
        grid=(N // B,),
        in_specs=[
            pl.BlockSpec((B, H), lambda i: (i, 0)),
            pl.BlockSpec((NC, 1, B, H), lambda i: (0, 0, i, 0)),
            full((H, H)), full((H, H)), full((H,)),
        ],
        out_specs=pl.BlockSpec((B, H), lambda i: (i, 0)),
        out_shape=jax.ShapeDtypeStruct((N, H), jnp.float32),
    )(x_emb, scat, W1, W2, b)


def _node_dense_kernel(x_ref, vec_ref, lng_ref, lnb_ref, qW_ref, qb_ref,
                       kW_ref, kb_ref, vWx_ref, vbx_ref, vW1_ref, vb1_ref,
                       vW2_ref, vb2_ref, vecW_ref,
                       q_ref, kvv_ref, vecdot_ref, vec3_ref):
    x = x_ref[...]
    vec = vec_ref[...]
    xn = _ln_block(x, lng_ref[...], lnb_ref[...])
    q_ref[...] = jnp.dot(xn, qW_ref[...], preferred_element_type=jnp.float32) + qb_ref[...]
    kk = jnp.dot(xn, kW_ref[...], preferred_element_type=jnp.float32) + kb_ref[...]
    vx = jnp.dot(xn, vWx_ref[...], preferred_element_type=jnp.float32) + vbx_ref[...]
    v1 = jnp.dot(xn, vW1_ref[...], preferred_element_type=jnp.float32) + vb1_ref[...]
    v2 = jnp.dot(xn, vW2_ref[...], preferred_element_type=jnp.float32) + vb2_ref[...]
    kvv_ref[...] = jnp.concatenate(
        [kk, vx, v1, v2, vec, jnp.zeros_like(kk)], axis=1).astype(jnp.bfloat16)
    vecW = vecW_ref[...]
    vd = jnp.zeros_like(x)
    v3s = []
    for c in range(3):
        vp_c = jnp.dot(vec[:, c * H:(c + 1) * H], vecW,
                       preferred_element_type=jnp.float32)
        vd = vd + vp_c[:, 0:H] * vp_c[:, H:2 * H]
        v3s.append(vp_c[:, 2 * H:3 * H])
    vecdot_ref[...] = vd
    vec3_ref[...] = jnp.concatenate(v3s, axis=1)


def _node_dense(x, vec2d, lng, lnb, qW, qb, kW, kb, vWx, vbx, vW1, vb1,
                vW2, vb2, vecW):
    B = 1000
    full = lambda s: pl.BlockSpec(s, lambda i: tuple(0 for _ in s))
    out_shapes = (
        jax.ShapeDtypeStruct((N, H), jnp.float32),
        jax.ShapeDtypeStruct((N, 8 * H), jnp.bfloat16),
        jax.ShapeDtypeStruct((N, H), jnp.float32),
        jax.ShapeDtypeStruct((N, 3 * H), jnp.float32),
    )
    return pl.pallas_call(
        _node_dense_kernel,
        grid=(N // B,),
        in_specs=[
            pl.BlockSpec((B, H), lambda i: (i, 0)),
            pl.BlockSpec((B, 3 * H), lambda i: (i, 0)),
            full((H,)), full((H,)),
            full((H, H)), full((H,)),
            full((H, H)), full((H,)),
            full((H, H)), full((H,)),
            full((H, H)), full((H,)),
            full((H, H)), full((H,)),
            full((H, 3 * H)),
        ],
        out_specs=[
            pl.BlockSpec((B, H), lambda i: (i, 0)),
            pl.BlockSpec((B, 8 * H), lambda i: (i, 0)),
            pl.BlockSpec((B, H), lambda i: (i, 0)),
            pl.BlockSpec((B, 3 * H), lambda i: (i, 0)),
        ],
        out_shape=out_shapes,
    )(x, vec2d, lng, lnb, qW, qb, kW, kb, vWx, vbx, vW1, vb1, vW2, vb2, vecW)


def _edge_layer_kernel(geo_ref, qd_ref, kvv_ref,
                       dkW_ref, dkb_ref, dWx_ref, dbx_ref, dW1_ref, db1_ref,
                       dW2_ref, db2_ref, selT_ref, selB_ref, out_ref):
    geo = geo_ref[...]
    attr = geo[:, 0:NRBF_P]
    cc = geo[:, NRBF_P + 1:NRBF_P + 2]
    d_ij = geo[:, NRBF_P + 2:NRBF_P + 5]
    kvv = kvv_ref[...].astype(jnp.float32)
    k_s = kvv[:, 0:H]
    vx_s = kvv[:, H:2 * H]
    v1_s = kvv[:, 2 * H:3 * H]
    v2_s = kvv[:, 3 * H:4 * H]
    dk = _silu(jnp.dot(attr, dkW_ref[...], preferred_element_type=jnp.float32) + dkb_ref[...])
    dvx = _silu(jnp.dot(attr, dWx_ref[...], preferred_element_type=jnp.float32) + dbx_ref[...])
    dv1 = _silu(jnp.dot(attr, dW1_ref[...], preferred_element_type=jnp.float32) + db1_ref[...])
    dv2 = _silu(jnp.dot(attr, dW2_ref[...], preferred_element_type=jnp.float32) + db2_ref[...])
    prod = qd_ref[...].astype(jnp.float32) * k_s * dk
    attn = jnp.dot(prod, selT_ref[...], preferred_element_type=jnp.float32)
    attn = _silu(attn) * cc
    attnb = jnp.dot(attn, selB_ref[...], preferred_element_type=jnp.float32)
    xm = vx_s * dvx * attnb
    vm1 = v1_s * dv1 * attnb
    vm2 = v2_s * dv2 * attnb
    outs = [xm]
    for c in range(3):
        vec_c = kvv[:, (4 + c) * H:(5 + c) * H]
        outs.append(vec_c * vm1 + vm2 * d_ij[:, c:c + 1])
    out_ref[...] = jnp.concatenate(outs, axis=1)


def _edge_layer(geo, q_dst, kvv_src, dkW, dkb, dWx, dbx, dW1, db1,
                dW2, db2, selT, selB):
    B = 1024
    full = lambda s: pl.BlockSpec(s, lambda i: tuple(0 for _ in s))
    return pl.pallas_call(
        _edge_layer_kernel,
        grid=(EP // B,),
        in_specs=[
            pl.BlockSpec((B, H), lambda i: (i, 0)),
            pl.BlockSpec((B, H), lambda i: (i, 0)),
            pl.BlockSpec((B, 8 * H), lambda i: (i, 0)),
            full((NRBF_P, H)), full((H,)),
            full((NRBF_P, H)), full((H,)),
            full((NRBF_P, H)), full((H,)),
            full((NRBF_P, H)), full((H,)),
            full((H, NH)), full((NH, H)),
        ],
        out_specs=pl.BlockSpec((B, 4 * H), lambda i: (i, 0)),
        out_shape=jax.ShapeDtypeStruct((EP, 4 * H), jnp.float32),
    )(geo, q_dst, kvv_src, dkW, dkb, dWx, dbx, dW1, db1, dW2, db2, selT, selB)


def _epi_kernel(x_ref, vec_ref, vd_ref, v3_ref, sc_ref, oW_ref, ob_ref,
                x_out, vec_out):
    x_agg = sc_ref[0, 0] + sc_ref[1, 0]
    vec_agg = jnp.concatenate(
        [sc_ref[0, 1 + c] + sc_ref[1, 1 + c] for c in range(3)], axis=1)
    o = jnp.dot(x_agg, oW_ref[...], preferred_element_type=jnp.float32) + ob_ref[...]
    o1 = o[:, 0:H]
    o2 = o[:, H:2 * H]
    o3 = o[:, 2 * H:3 * H]
    x_out[...] = x_ref[...] + vd_ref[...] * o2 + o3
    o1t = jnp.concatenate([o1, o1, o1], axis=1)
    vec_out[...] = vec_ref[...] + v3_ref[...] * o1t + vec_agg


def _epilogue(x, vec2d, vec_dot, vec3, scat, oW, ob):
    B = 1000
    full = lambda s: pl.BlockSpec(s, lambda i: tuple(0 for _ in s))
    return pl.pallas_call(
        _epi_kernel,
        grid=(N // B,),
        in_specs=[
            pl.BlockSpec((B, H), lambda i: (i, 0)),
            pl.BlockSpec((B, 3 * H), lambda i: (i, 0)),
            pl.BlockSpec((B, H), lambda i: (i, 0)),
            pl.BlockSpec((B, 3 * H), lambda i: (i, 0)),
            pl.BlockSpec((NC, 4, B, H), lambda i: (0, 0, i, 0)),
            full((H, 3 * H)), full((3 * H,)),
        ],
        out_specs=[
            pl.BlockSpec((B, H), lambda i: (i, 0)),
            pl.BlockSpec((B, 3 * H), lambda i: (i, 0)),
        ],
        out_shape=(
            jax.ShapeDtypeStruct((N, H), jnp.float32),
            jax.ShapeDtypeStruct((N, 3 * H), jnp.float32),
        ),
    )(x, vec2d, vec_dot, vec3, scat, oW, ob)


def _ln_kernel(x_ref, g_ref, b_ref, out_ref):
    out_ref[...] = _ln_block(x_ref[...], g_ref[...], b_ref[...])


def _final_ln(x, g, b):
    B = 1000
    full = lambda s: pl.BlockSpec(s, lambda i: tuple(0 for _ in s))
    return pl.pallas_call(
        _ln_kernel,
        grid=(N // B,),
        in_specs=[
            pl.BlockSpec((B, H), lambda i: (i, 0)),
            full((H,)), full((H,)),
        ],
        out_specs=pl.BlockSpec((B, H), lambda i: (i, 0)),
        out_shape=jax.ShapeDtypeStruct((N, H), jnp.float32),
    )(x, g, b)


def kernel(pos, elems, edge_index, emb_table, ne_emb_table, ne_dist_W,
           ne_dist_b, ne_comb_W, ne_comb_b, rbf_means, rbf_betas, ln_g, ln_b,
           qW, qb, kW, kb, vW, vb, oW, ob, vecW, dkW, dkb, dvW, dvb,
           outn_g, outn_b):
    f32 = jnp.float32
    src = edge_index[0].astype(jnp.int32)
    dst = edge_index[1].astype(jnp.int32)
    src_p = jnp.pad(src, (0, EP - E))
    dst_p = jnp.pad(dst, (0, EP - E))
    elems_p = jnp.pad(elems.astype(jnp.int32), (0, NP - N))

    ptab = jnp.concatenate(
        [pos.astype(f32), jnp.arange(N, dtype=f32)[:, None],
         jnp.zeros((N, H - 4), f32)], axis=1)
    means = jnp.pad(rbf_means, (0, NRBF_P - NRBF))
    betas = jnp.pad(rbf_betas, (0, NRBF_P - NRBF))
    ne_dist_Wp = jnp.pad(ne_dist_W, ((0, NRBF_P - NRBF), (0, 0)))

    selT = (jnp.arange(H)[:, None] // HD == jnp.arange(NH)[None, :]).astype(f32)
    selB = selT.T

    def _perm_v(W, b):
        Wr = W.reshape(-1, NH, 3, HD)
        br = b.reshape(NH, 3, HD)
        Ws = [Wr[:, :, s, :].reshape(-1, H) for s in range(3)]
        bs = [br[:, s, :].reshape(H) for s in range(3)]
        return Ws, bs

    zeros_acc = jnp.zeros((NACC, H), f32)

    x_emb = _sc_gather(emb_table, elems_p, H)[:N]
    xz = _sc_gather(ne_emb_table, elems_p, H)[:N]
    ps = _sc_gather(ptab, src_p, H)
    pd = _sc_gather(ptab, dst_p, H)
    geo = _geo_edge(ps, pd, means, betas)
    xz_src = _sc_gather(xz, src_p, H)
    ne_msg = _ne_edge(geo, xz_src, ne_dist_Wp, ne_dist_b)
    ne_scat = _sc_scatter(ne_msg, dst_p, zeros_acc, 1)
    x = _ne_comb(x_emb, ne_scat, ne_comb_W[0:H], ne_comb_W[H:2 * H], ne_comb_b)

    vec2d = jnp.zeros((N, 3 * H), f32)
    for l in range(L):
        (vWs, vbs) = _perm_v(vW[l], vb[l])
        (dWs, dbs) = _perm_v(jnp.pad(dvW[l], ((0, NRBF_P - NRBF), (0, 0))),
                             dvb[l])
        dkWp = jnp.pad(dkW[l], ((0, NRBF_P - NRBF), (0, 0)))
        q, kvv, vec_dot, vec3 = _node_dense(
            x, vec2d, ln_g[l], ln_b[l], qW[l], qb[l], kW[l], kb[l],
            vWs[0], vbs[0], vWs[1], vbs[1], vWs[2], vbs[2], vecW[l])
        kvv32 = lax.bitcast_convert_type(kvv.reshape(N, 4 * H, 2), jnp.int32)
        q_dst = _sc_gather(q, dst_p, H)
        kvv_src = lax.bitcast_convert_type(
            _sc_gather(kvv32, src_p, 4 * H), jnp.bfloat16).reshape(EP, 8 * H)
        msgs = _edge_layer(geo, q_dst, kvv_src,
                           dkWp, dkb[l], dWs[0], dbs[0], dWs[1], dbs[1],
                           dWs[2], dbs[2], selT, selB)
        scat = _sc_scatter(msgs, dst_p, zeros_acc, 4)
        x, vec2d = _epilogue(x, vec2d, vec_dot, vec3, scat, oW[l], ob[l])

    x = _final_ln(x, outn_g, outn_b)
    return x, vec2d.reshape(N, 3, H)


# speedup vs baseline: 2.2597x; 2.2597x over previous
import functools
import jax
import jax.numpy as jnp
from jax import lax
from jax.experimental import pallas as pl
from jax.experimental.pallas import tpu as pltpu
from jax.experimental.pallas import tpu_sc as plsc

N = 10000
E = 160000
H = 128
NH = 8
HD = 16
NRBF = 50
NRBF_P = 64
L = 2
MAX_Z = 100
CUTOFF = 5.0

NC = 2
NS = 16
NW = NC * NS
CH = 128
EP = 163840
NP = 12288
NACC = 10240
ROWS_S = NACC // NS


def _silu(x):
    return x * jax.nn.sigmoid(x)


def _ln_block(x, g, b):
    m = jnp.mean(x, axis=-1, keepdims=True)
    v = jnp.mean((x - m) ** 2, axis=-1, keepdims=True)
    return (x - m) / jnp.sqrt(v + 1e-5) * g + b


def _sc_gather(table, idx, D):
    EPi = idx.shape[0]
    per_w = EPi // NW
    dt = table.dtype
    nbytes = D * dt.itemsize
    GB = 64 if nbytes >= 2048 else CH
    NB = 2 if nbytes >= 2048 else 4
    nch = per_w // GB
    mesh = plsc.VectorSubcoreMesh(core_axis_name="c", subcore_axis_name="s")

    @functools.partial(
        pl.kernel,
        mesh=mesh,
        out_type=jax.ShapeDtypeStruct((EPi, D), dt),
        scratch_types=[
            pltpu.VMEM((per_w,), jnp.int32),
            pltpu.VMEM((NB, GB, D), dt),
        ] + [pltpu.SemaphoreType.DMA] * (2 * NB),
    )
    def k(table_hbm, idx_hbm, out_hbm, idx_v, rows_v, *sems):
        gsem = sems[:NB]
        ssem = sems[NB:]
        wid = lax.axis_index("s") * NC + lax.axis_index("c")
        base = wid * per_w
        pltpu.sync_copy(idx_hbm.at[pl.ds(base, per_w)], idx_v)

        def gather_cp(j, b):
            return pltpu.make_async_copy(table_hbm.at[idx_v.at[pl.ds(j * GB, GB)]],
                                         rows_v.at[b], gsem[b])

        def store_cp(j, b):
            return pltpu.make_async_copy(
                rows_v.at[b], out_hbm.at[pl.ds(base + j * GB, GB)], ssem[b])

        def step(j, b):
            nxt = j + NB - 1

            @pl.when(nxt < nch)
            def _():
                gather_cp(nxt, (b + NB - 1) % NB).start()

            gather_cp(j, b).wait()
            cp = store_cp(j, b)
            cp.start()
            cp.wait()

        for p in range(NB - 1):
            if p < nch:
                gather_cp(p, p).start()

        def body(jo, carry):
            for b in range(NB):
                step(NB * jo + b, b)
            return carry

        lax.fori_loop(0, nch // NB, body, 0)
        for r in range(nch - (nch % NB), nch):
            step(r, r % NB)

    return k(table, idx)


def _sc_scatter(msgs, idx, zeros, C):
    EPi = idx.shape[0]
    per_w = EPi // NW
    nch = per_w // CH
    idx2 = idx.reshape(-1, CH)
    mesh = plsc.VectorSubcoreMesh(core_axis_name="c", subcore_axis_name="s")

    @functools.partial(
        pl.kernel,
        mesh=mesh,
        out_type=jax.ShapeDtypeStruct((NC, C, NACC, H), jnp.float32),
        scratch_types=[
            pltpu.VMEM((nch, CH), jnp.int32),
            pltpu.VMEM((2, CH, H), jnp.float32),
            pltpu.VMEM_SHARED((NACC, H), jnp.float32),
            pltpu.SemaphoreType.DMA,
            pltpu.SemaphoreType.DMA,
        ],
    )
    def k(msgs_hbm, idx_hbm, zeros_hbm, out_hbm, idx_v, msg_v, acc, l0, l1):
        lsem = (l0, l1)
        cid = lax.axis_index("c")
        sid = lax.axis_index("s")
        wid = sid * NC + cid
        base = wid * per_w
        pltpu.sync_copy(idx_hbm.at[pl.ds(wid * nch, nch)], idx_v)
        for c in range(C):
            pltpu.sync_copy(zeros_hbm.at[pl.ds(sid * ROWS_S, ROWS_S)],
                            acc.at[pl.ds(sid * ROWS_S, ROWS_S)])
            plsc.subcore_barrier()

            def load_cp(j, b):
                return pltpu.make_async_copy(
                    msgs_hbm.at[pl.ds(base + j * CH, CH), pl.ds(c * H, H)],
                    msg_v.at[b], lsem[b])

            def step(j, b):
                nxt = j + 1

                @pl.when(nxt < nch)
                def _():
                    load_cp(nxt, 1 - b).start()

                load_cp(j, b).wait()
                pltpu.sync_copy(msg_v.at[b], acc.at[idx_v.at[j]], add=True)

            load_cp(0, 0).start()

            def body(jo, carry):
                step(2 * jo, 0)
                step(2 * jo + 1, 1)
                return carry

            lax.fori_loop(0, nch // 2, body, 0)
            if nch % 2:
                step(nch - 1, (nch - 1) % 2)
            plsc.subcore_barrier()
            pltpu.sync_copy(acc.at[pl.ds(sid * ROWS_S, ROWS_S)],
                            out_hbm.at[cid, c, pl.ds(sid * ROWS_S, ROWS_S)])
            plsc.subcore_barrier()

    return k(msgs, idx2, zeros)


def _geo_kernel(ps_ref, pd_ref, means_ref, betas_ref, out_ref):
    B = ps_ref.shape[0]
    ps = ps_ref[...]
    pd = pd_ref[...]
    pv = pd[:, 0:3] - ps[:, 0:3]
    d2 = jnp.sum(pv * pv, axis=1, keepdims=True)
    w = jnp.sqrt(jnp.maximum(d2, 1e-12))
    attr = jnp.exp(-betas_ref[...] * (jnp.exp(-w) - means_ref[...]) ** 2)
    cc = 0.5 * (jnp.cos(w * (jnp.pi / CUTOFF)) + 1.0) * (w < CUTOFF).astype(jnp.float32)
    d_ij = pv / w
    ids = pl.program_id(0) * B + lax.broadcasted_iota(jnp.int32, (B, 1), 0)
    padf = (ids < E).astype(jnp.float32)
    mask = (ps[:, 3:4] != pd[:, 3:4]).astype(jnp.float32)
    out_ref[...] = jnp.concatenate(
        [attr, cc * mask * padf, cc * padf, d_ij,
         jnp.zeros((B, H - NRBF_P - 5), jnp.float32)], axis=1)


def _geo_edge(pp, means, betas):
    B = 2048
    nb = EP // B
    full = lambda s: pl.BlockSpec(s, lambda i: tuple(0 for _ in s))
    return pl.pallas_call(
        _geo_kernel,
        grid=(nb,),
        in_specs=[
            pl.BlockSpec((B, H), lambda i: (i, 0)),
            pl.BlockSpec((B, H), lambda i: (i + nb, 0)),
            full((NRBF_P,)), full((NRBF_P,)),
        ],
        out_specs=pl.BlockSpec((B, H), lambda i: (i, 0)),
        out_shape=jax.ShapeDtypeStruct((EP, H), jnp.float32),
    )(pp, pp, means, betas)


def _ne_edge_kernel(geo_ref, xz_ref, W_ref, b_ref, out_ref):
    geo = geo_ref[...]
    attr = geo[:, 0:NRBF_P]
    wmsg = jnp.dot(attr, W_ref[...], preferred_element_type=jnp.float32) + b_ref[...]
    out_ref[...] = xz_ref[...] * wmsg * geo[:, NRBF_P:NRBF_P + 1]


def _ne_edge(geo, pp, W, b):
    B = 2048
    nb = EP // B
    full = lambda s: pl.BlockSpec(s, lambda i: tuple(0 for _ in s))
    return pl.pallas_call(
        _ne_edge_kernel,
        grid=(nb,),
        in_specs=[
            pl.BlockSpec((B, H), lambda i: (i, 0)),
            pl.BlockSpec((B, H), lambda i: (i + 2 * nb, 0)),
            full((NRBF_P, H)), full((H,)),
        ],
        out_specs=pl.BlockSpec((B, H), lambda i: (i, 0)),
        out_shape=jax.ShapeDtypeStruct((EP, H), jnp.float32),
    )(geo, pp, W, b)


def _ne_comb_kernel(xe_ref, sc_ref, W1_ref, W2_ref, b_ref, out_ref):
    agg = sc_ref[0, 0] + sc_ref[1, 0]
    out_ref[...] = (jnp.dot(xe_ref[...], W1_ref[...], preferred_element_type=jnp.float32)
                    + jnp.dot(agg, W2_ref[...], preferred_element_type=jnp.float32)
                    + b_ref[...])


def _ne_comb(x_emb, scat, W1, W2, b):
    B = 1000
    full = lambda s: pl.BlockSpec(s, lambda i: tuple(0 for _ in s))
    return pl.pallas_call(
        _ne_comb_kernel,
        grid=(N // B,),
        in_specs=[
            pl.BlockSpec((B, H), lambda i: (i, 0)),
            pl.BlockSpec((NC, 1, B, H), lambda i: (0, 0, i, 0)),
            full((H, H)), full((H, H)), full((H,)),
        ],
        out_specs=pl.BlockSpec((B, H), lambda i: (i, 0)),
        out_shape=jax.ShapeDtypeStruct((N, H), jnp.float32),
    )(x_emb, scat, W1, W2, b)


def _node_dense_kernel(x_ref, vec_ref, lng_ref, lnb_ref, qW_ref, qb_ref,
                       kW_ref, kb_ref, vWx_ref, vbx_ref, vW1_ref, vb1_ref,
                       vW2_ref, vb2_ref, vecW_ref,
                       q_ref, kvv_ref, vecdot_ref, vec3_ref):
    x = x_ref[...]
    vec = vec_ref[...]
    xn = _ln_block(x, lng_ref[...], lnb_ref[...])
    q_ref[...] = jnp.dot(xn, qW_ref[...], preferred_element_type=jnp.float32) + qb_ref[...]
    kk = jnp.dot(xn, kW_ref[...], preferred_element_type=jnp.float32) + kb_ref[...]
    vx = jnp.dot(xn, vWx_ref[...], preferred_element_type=jnp.float32) + vbx_ref[...]
    v1 = jnp.dot(xn, vW1_ref[...], preferred_element_type=jnp.float32) + vb1_ref[...]
    v2 = jnp.dot(xn, vW2_ref[...], preferred_element_type=jnp.float32) + vb2_ref[...]
    kvv_ref[...] = jnp.concatenate([kk, vx, v1, v2, vec], axis=1)
    vecW = vecW_ref[...]
    vd = jnp.zeros_like(x)
    v3s = []
    for c in range(3):
        vp_c = jnp.dot(vec[:, c * H:(c + 1) * H], vecW,
                       preferred_element_type=jnp.float32)
        vd = vd + vp_c[:, 0:H] * vp_c[:, H:2 * H]
        v3s.append(vp_c[:, 2 * H:3 * H])
    vecdot_ref[...] = vd
    vec3_ref[...] = jnp.concatenate(v3s, axis=1)


def _node_dense(x, vec2d, lng, lnb, qW, qb, kW, kb, vWx, vbx, vW1, vb1,
                vW2, vb2, vecW):
    B = 1000
    full = lambda s: pl.BlockSpec(s, lambda i: tuple(0 for _ in s))
    out_shapes = (
        jax.ShapeDtypeStruct((N, H), jnp.float32),
        jax.ShapeDtypeStruct((N, 7 * H), jnp.float32),
        jax.ShapeDtypeStruct((N, H), jnp.float32),
        jax.ShapeDtypeStruct((N, 3 * H), jnp.float32),
    )
    return pl.pallas_call(
        _node_dense_kernel,
        grid=(N // B,),
        in_specs=[
            pl.BlockSpec((B, H), lambda i: (i, 0)),
            pl.BlockSpec((B, 3 * H), lambda i: (i, 0)),
            full((H,)), full((H,)),
            full((H, H)), full((H,)),
            full((H, H)), full((H,)),
            full((H, H)), full((H,)),
            full((H, H)), full((H,)),
            full((H, H)), full((H,)),
            full((H, 3 * H)),
        ],
        out_specs=[
            pl.BlockSpec((B, H), lambda i: (i, 0)),
            pl.BlockSpec((B, 7 * H), lambda i: (i, 0)),
            pl.BlockSpec((B, H), lambda i: (i, 0)),
            pl.BlockSpec((B, 3 * H), lambda i: (i, 0)),
        ],
        out_shape=out_shapes,
    )(x, vec2d, lng, lnb, qW, qb, kW, kb, vWx, vbx, vW1, vb1, vW2, vb2, vecW)


def _edge_layer_kernel(geo_ref, qd_ref, kvv_ref,
                       dkW_ref, dkb_ref, dWx_ref, dbx_ref, dW1_ref, db1_ref,
                       dW2_ref, db2_ref, selT_ref, selB_ref, out_ref):
    geo = geo_ref[...]
    attr = geo[:, 0:NRBF_P]
    cc = geo[:, NRBF_P + 1:NRBF_P + 2]
    d_ij = geo[:, NRBF_P + 2:NRBF_P + 5]
    kvv = kvv_ref[...].astype(jnp.float32)
    k_s = kvv[:, 0:H]
    vx_s = kvv[:, H:2 * H]
    v1_s = kvv[:, 2 * H:3 * H]
    v2_s = kvv[:, 3 * H:4 * H]
    dk = _silu(jnp.dot(attr, dkW_ref[...], preferred_element_type=jnp.float32) + dkb_ref[...])
    dvx = _silu(jnp.dot(attr, dWx_ref[...], preferred_element_type=jnp.float32) + dbx_ref[...])
    dv1 = _silu(jnp.dot(attr, dW1_ref[...], preferred_element_type=jnp.float32) + db1_ref[...])
    dv2 = _silu(jnp.dot(attr, dW2_ref[...], preferred_element_type=jnp.float32) + db2_ref[...])
    prod = qd_ref[...].astype(jnp.float32) * k_s * dk
    attn = jnp.dot(prod, selT_ref[...], preferred_element_type=jnp.float32)
    attn = _silu(attn) * cc
    attnb = jnp.dot(attn, selB_ref[...], preferred_element_type=jnp.float32)
    xm = vx_s * dvx * attnb
    vm1 = v1_s * dv1 * attnb
    vm2 = v2_s * dv2 * attnb
    outs = [xm]
    for c in range(3):
        vec_c = kvv[:, (4 + c) * H:(5 + c) * H]
        outs.append(vec_c * vm1 + vm2 * d_ij[:, c:c + 1])
    out_ref[...] = jnp.concatenate(outs, axis=1)


def _edge_layer(geo, q_dst, kvv_src, dkW, dkb, dWx, dbx, dW1, db1,
                dW2, db2, selT, selB):
    B = 1024
    full = lambda s: pl.BlockSpec(s, lambda i: tuple(0 for _ in s))
    return pl.pallas_call(
        _edge_layer_kernel,
        grid=(EP // B,),
        in_specs=[
            pl.BlockSpec((B, H), lambda i: (i, 0)),
            pl.BlockSpec((B, H), lambda i: (i, 0)),
            pl.BlockSpec((B, 7 * H), lambda i: (i, 0)),
            full((NRBF_P, H)), full((H,)),
            full((NRBF_P, H)), full((H,)),
            full((NRBF_P, H)), full((H,)),
            full((NRBF_P, H)), full((H,)),
            full((H, NH)), full((NH, H)),
        ],
        out_specs=pl.BlockSpec((B, 4 * H), lambda i: (i, 0)),
        out_shape=jax.ShapeDtypeStruct((EP, 4 * H), jnp.float32),
    )(geo, q_dst, kvv_src, dkW, dkb, dWx, dbx, dW1, db1, dW2, db2, selT, selB)


def _epi_kernel(x_ref, vec_ref, vd_ref, v3_ref, sc_ref, oW_ref, ob_ref,
                x_out, vec_out):
    x_agg = sc_ref[0, 0] + sc_ref[1, 0]
    vec_agg = jnp.concatenate(
        [sc_ref[0, 1 + c] + sc_ref[1, 1 + c] for c in range(3)], axis=1)
    o = jnp.dot(x_agg, oW_ref[...], preferred_element_type=jnp.float32) + ob_ref[...]
    o1 = o[:, 0:H]
    o2 = o[:, H:2 * H]
    o3 = o[:, 2 * H:3 * H]
    x_out[...] = x_ref[...] + vd_ref[...] * o2 + o3
    o1t = jnp.concatenate([o1, o1, o1], axis=1)
    vec_out[...] = vec_ref[...] + v3_ref[...] * o1t + vec_agg


def _epilogue(x, vec2d, vec_dot, vec3, scat, oW, ob):
    B = 1000
    full = lambda s: pl.BlockSpec(s, lambda i: tuple(0 for _ in s))
    return pl.pallas_call(
        _epi_kernel,
        grid=(N // B,),
        in_specs=[
            pl.BlockSpec((B, H), lambda i: (i, 0)),
            pl.BlockSpec((B, 3 * H), lambda i: (i, 0)),
            pl.BlockSpec((B, H), lambda i: (i, 0)),
            pl.BlockSpec((B, 3 * H), lambda i: (i, 0)),
            pl.BlockSpec((NC, 4, B, H), lambda i: (0, 0, i, 0)),
            full((H, 3 * H)), full((3 * H,)),
        ],
        out_specs=[
            pl.BlockSpec((B, H), lambda i: (i, 0)),
            pl.BlockSpec((B, 3 * H), lambda i: (i, 0)),
        ],
        out_shape=(
            jax.ShapeDtypeStruct((N, H), jnp.float32),
            jax.ShapeDtypeStruct((N, 3 * H), jnp.float32),
        ),
    )(x, vec2d, vec_dot, vec3, scat, oW, ob)


def _ln_kernel(x_ref, g_ref, b_ref, out_ref):
    out_ref[...] = _ln_block(x_ref[...], g_ref[...], b_ref[...])


def _final_ln(x, g, b):
    B = 1000
    full = lambda s: pl.BlockSpec(s, lambda i: tuple(0 for _ in s))
    return pl.pallas_call(
        _ln_kernel,
        grid=(N // B,),
        in_specs=[
            pl.BlockSpec((B, H), lambda i: (i, 0)),
            full((H,)), full((H,)),
        ],
        out_specs=pl.BlockSpec((B, H), lambda i: (i, 0)),
        out_shape=jax.ShapeDtypeStruct((N, H), jnp.float32),
    )(x, g, b)


def kernel(pos, elems, edge_index, emb_table, ne_emb_table, ne_dist_W,
           ne_dist_b, ne_comb_W, ne_comb_b, rbf_means, rbf_betas, ln_g, ln_b,
           qW, qb, kW, kb, vW, vb, oW, ob, vecW, dkW, dkb, dvW, dvb,
           outn_g, outn_b):
    f32 = jnp.float32
    src = edge_index[0].astype(jnp.int32)
    dst = edge_index[1].astype(jnp.int32)
    src_p = jnp.pad(src, (0, EP - E))
    dst_p = jnp.pad(dst, (0, EP - E))
    elems_p = jnp.pad(elems.astype(jnp.int32), (0, NP - N))

    ptab = jnp.concatenate(
        [pos.astype(f32), jnp.arange(N, dtype=f32)[:, None],
         jnp.zeros((N, H - 4), f32)], axis=1)
    means = jnp.pad(rbf_means, (0, NRBF_P - NRBF))
    betas = jnp.pad(rbf_betas, (0, NRBF_P - NRBF))
    ne_dist_Wp = jnp.pad(ne_dist_W, ((0, NRBF_P - NRBF), (0, 0)))

    selT = (jnp.arange(H)[:, None] // HD == jnp.arange(NH)[None, :]).astype(f32)
    selB = selT.T

    def _perm_v(W, b):
        Wr = W.reshape(-1, NH, 3, HD)
        br = b.reshape(NH, 3, HD)
        Ws = [Wr[:, :, s, :].reshape(-1, H) for s in range(3)]
        bs = [br[:, s, :].reshape(H) for s in range(3)]
        return Ws, bs

    zeros_acc = jnp.zeros((NACC, H), f32)

    embcat = jnp.concatenate([emb_table, ne_emb_table], axis=0)
    elems2 = jnp.concatenate([elems_p, elems_p + MAX_Z])
    emb2 = _sc_gather(embcat, elems2, H)
    x_emb = emb2[:N]
    ptab2 = jnp.concatenate([ptab, emb2[NP:NP + N]], axis=0)
    idx3 = jnp.concatenate([src_p, dst_p, src_p + N])
    pp = _sc_gather(ptab2, idx3, H)
    geo = _geo_edge(pp, means, betas)
    ne_msg = _ne_edge(geo, pp, ne_dist_Wp, ne_dist_b)
    ne_scat = _sc_scatter(ne_msg, dst_p, zeros_acc, 1)
    x = _ne_comb(x_emb, ne_scat, ne_comb_W[0:H], ne_comb_W[H:2 * H], ne_comb_b)

    vec2d = jnp.zeros((N, 3 * H), f32)
    for l in range(L):
        (vWs, vbs) = _perm_v(vW[l], vb[l])
        (dWs, dbs) = _perm_v(jnp.pad(dvW[l], ((0, NRBF_P - NRBF), (0, 0))),
                             dvb[l])
        dkWp = jnp.pad(dkW[l], ((0, NRBF_P - NRBF), (0, 0)))
        q, kvv, vec_dot, vec3 = _node_dense(
            x, vec2d, ln_g[l], ln_b[l], qW[l], qb[l], kW[l], kb[l],
            vWs[0], vbs[0], vWs[1], vbs[1], vWs[2], vbs[2], vecW[l])
        q_dst = _sc_gather(q, dst_p, H)
        kvv_src = _sc_gather(kvv, src_p, 7 * H)
        msgs = _edge_layer(geo, q_dst, kvv_src,
                           dkWp, dkb[l], dWs[0], dbs[0], dWs[1], dbs[1],
                           dWs[2], dbs[2], selT, selB)
        scat = _sc_scatter(msgs, dst_p, zeros_acc, 4)
        x, vec2d = _epilogue(x, vec2d, vec_dot, vec3, scat, oW[l], ob[l])

    x = _final_ln(x, outn_g, outn_b)
    return x, vec2d.reshape(N, 3, H)


# speedup vs baseline: 2.4937x; 1.1035x over previous
import functools
import jax
import jax.numpy as jnp
from jax import lax
from jax.experimental import pallas as pl
from jax.experimental.pallas import tpu as pltpu
from jax.experimental.pallas import tpu_sc as plsc

N = 10000
E = 160000
H = 128
NH = 8
HD = 16
NRBF = 50
NRBF_P = 64
L = 2
MAX_Z = 100
CUTOFF = 5.0

NC = 2
NS = 16
NW = NC * NS
CH = 128
EP = 163840
NP = 12288
NACC = 10240
ROWS_S = NACC // NS


def _silu(x):
    return x * jax.nn.sigmoid(x)


def _ln_block(x, g, b):
    m = jnp.mean(x, axis=-1, keepdims=True)
    v = jnp.mean((x - m) ** 2, axis=-1, keepdims=True)
    return (x - m) / jnp.sqrt(v + 1e-5) * g + b


def _sc_gather(table, idx, D):
    EPi = idx.shape[0]
    per_w = EPi // NW
    dt = table.dtype
    nbytes = D * dt.itemsize
    GB = 64 if nbytes >= 2048 else CH
    NB = 2 if nbytes >= 2048 else 4
    nch = per_w // GB
    mesh = plsc.VectorSubcoreMesh(core_axis_name="c", subcore_axis_name="s")

    @functools.partial(
        pl.kernel,
        mesh=mesh,
        out_type=jax.ShapeDtypeStruct((EPi, D), dt),
        scratch_types=[
            pltpu.VMEM((per_w,), jnp.int32),
            pltpu.VMEM((NB, GB, D), dt),
        ] + [pltpu.SemaphoreType.DMA] * (2 * NB),
    )
    def k(table_hbm, idx_hbm, out_hbm, idx_v, rows_v, *sems):
        gsem = sems[:NB]
        ssem = sems[NB:]
        wid = lax.axis_index("s") * NC + lax.axis_index("c")
        base = wid * per_w
        pltpu.sync_copy(idx_hbm.at[pl.ds(base, per_w)], idx_v)

        def gather_cp(j, b):
            return pltpu.make_async_copy(table_hbm.at[idx_v.at[pl.ds(j * GB, GB)]],
                                         rows_v.at[b], gsem[b])

        def store_cp(j, b):
            return pltpu.make_async_copy(
                rows_v.at[b], out_hbm.at[pl.ds(base + j * GB, GB)], ssem[b])

        def step(j, b):
            nxt = j + NB - 1

            @pl.when(nxt < nch)
            def _():
                gather_cp(nxt, (b + NB - 1) % NB).start()

            gather_cp(j, b).wait()
            cp = store_cp(j, b)
            cp.start()
            cp.wait()

        for p in range(NB - 1):
            if p < nch:
                gather_cp(p, p).start()

        def body(jo, carry):
            for b in range(NB):
                step(NB * jo + b, b)
            return carry

        lax.fori_loop(0, nch // NB, body, 0)
        for r in range(nch - (nch % NB), nch):
            step(r, r % NB)

    return k(table, idx)


def _sc_scatter(msgs, idx, zeros, C):
    EPi = idx.shape[0]
    per_w = EPi // NW
    nch = per_w // CH
    idx2 = idx.reshape(-1, CH)
    mesh = plsc.VectorSubcoreMesh(core_axis_name="c", subcore_axis_name="s")

    @functools.partial(
        pl.kernel,
        mesh=mesh,
        out_type=jax.ShapeDtypeStruct((NC, C, NACC, H), jnp.float32),
        scratch_types=[
            pltpu.VMEM((nch, CH), jnp.int32),
            pltpu.VMEM((2, CH, H), jnp.float32),
            pltpu.VMEM_SHARED((NACC, H), jnp.float32),
            pltpu.SemaphoreType.DMA,
            pltpu.SemaphoreType.DMA,
        ],
    )
    def k(msgs_hbm, idx_hbm, zeros_hbm, out_hbm, idx_v, msg_v, acc, l0, l1):
        lsem = (l0, l1)
        cid = lax.axis_index("c")
        sid = lax.axis_index("s")
        wid = sid * NC + cid
        base = wid * per_w
        pltpu.sync_copy(idx_hbm.at[pl.ds(wid * nch, nch)], idx_v)
        for c in range(C):
            pltpu.sync_copy(zeros_hbm.at[pl.ds(sid * ROWS_S, ROWS_S)],
                            acc.at[pl.ds(sid * ROWS_S, ROWS_S)])
            plsc.subcore_barrier()

            def load_cp(j, b):
                return pltpu.make_async_copy(
                    msgs_hbm.at[pl.ds(base + j * CH, CH), pl.ds(c * H, H)],
                    msg_v.at[b], lsem[b])

            def step(j, b):
                nxt = j + 1

                @pl.when(nxt < nch)
                def _():
                    load_cp(nxt, 1 - b).start()

                load_cp(j, b).wait()
                pltpu.sync_copy(msg_v.at[b], acc.at[idx_v.at[j]], add=True)

            load_cp(0, 0).start()

            def body(jo, carry):
                step(2 * jo, 0)
                step(2 * jo + 1, 1)
                return carry

            lax.fori_loop(0, nch // 2, body, 0)
            if nch % 2:
                step(nch - 1, (nch - 1) % 2)
            plsc.subcore_barrier()
            pltpu.sync_copy(acc.at[pl.ds(sid * ROWS_S, ROWS_S)],
                            out_hbm.at[cid, c, pl.ds(sid * ROWS_S, ROWS_S)])
            plsc.subcore_barrier()

    return k(msgs, idx2, zeros)


def _geo_kernel(ps_ref, pd_ref, means_ref, betas_ref, out_ref):
    B = ps_ref.shape[0]
    ps = ps_ref[...]
    pd = pd_ref[...]
    pv = pd[:, 0:3] - ps[:, 0:3]
    d2 = jnp.sum(pv * pv, axis=1, keepdims=True)
    w = jnp.sqrt(jnp.maximum(d2, 1e-12))
    attr = jnp.exp(-betas_ref[...] * (jnp.exp(-w) - means_ref[...]) ** 2)
    cc = 0.5 * (jnp.cos(w * (jnp.pi / CUTOFF)) + 1.0) * (w < CUTOFF).astype(jnp.float32)
    d_ij = pv / w
    ids = pl.program_id(0) * B + lax.broadcasted_iota(jnp.int32, (B, 1), 0)
    padf = (ids < E).astype(jnp.float32)
    mask = (ps[:, 3:4] != pd[:, 3:4]).astype(jnp.float32)
    out_ref[...] = jnp.concatenate(
        [attr, cc * mask * padf, cc * padf, d_ij,
         jnp.zeros((B, H - NRBF_P - 5), jnp.float32)], axis=1)


def _geo_edge(pp, means, betas):
    B = 2048
    nb = EP // B
    full = lambda s: pl.BlockSpec(s, lambda i: tuple(0 for _ in s))
    return pl.pallas_call(
        _geo_kernel,
        grid=(nb,),
        in_specs=[
            pl.BlockSpec((B, H), lambda i: (i, 0)),
            pl.BlockSpec((B, H), lambda i: (i + nb, 0)),
            full((NRBF_P,)), full((NRBF_P,)),
        ],
        out_specs=pl.BlockSpec((B, H), lambda i: (i, 0)),
        out_shape=jax.ShapeDtypeStruct((EP, H), jnp.float32),
    )(pp, pp, means, betas)


def _ne_edge_kernel(geo_ref, xz_ref, W_ref, b_ref, out_ref):
    geo = geo_ref[...]
    attr = geo[:, 0:NRBF_P]
    wmsg = jnp.dot(attr, W_ref[...], preferred_element_type=jnp.float32) + b_ref[...]
    out_ref[...] = xz_ref[...] * wmsg * geo[:, NRBF_P:NRBF_P + 1]


def _ne_edge(geo, pp, W, b):
    B = 2048
    nb = EP // B
    full = lambda s: pl.BlockSpec(s, lambda i: tuple(0 for _ in s))
    return pl.pallas_call(
        _ne_edge_kernel,
        grid=(nb,),
        in_specs=[
            pl.BlockSpec((B, H), lambda i: (i, 0)),
            pl.BlockSpec((B, H), lambda i: (i + 2 * nb, 0)),
            full((NRBF_P, H)), full((H,)),
        ],
        out_specs=pl.BlockSpec((B, H), lambda i: (i, 0)),
        out_shape=jax.ShapeDtypeStruct((EP, H), jnp.float32),
    )(geo, pp, W, b)


def _ne_comb_kernel(xe_ref, sc_ref, W1_ref, W2_ref, b_ref, out_ref):
    agg = sc_ref[0, 0] + sc_ref[1, 0]
    out_ref[...] = (jnp.dot(xe_ref[...], W1_ref[...], preferred_element_type=jnp.float32)
                    + jnp.dot(agg, W2_ref[...], preferred_element_type=jnp.float32)
                    + b_ref[...])


def _ne_comb(x_emb, scat, W1, W2, b):
    B = 1000
    full = lambda s: pl.BlockSpec(s, lambda i: tuple(0 for _ in s))
    return pl.pallas_call(
        _ne_comb_kernel,
        grid=(N // B,),
        in_specs=[
            pl.BlockSpec((B, H), lambda i: (i, 0)),
            pl.BlockSpec((NC, 1, B, H), lambda i: (0, 0, i, 0)),
            full((H, H)), full((H, H)), full((H,)),
        ],
        out_specs=pl.BlockSpec((B, H), lambda i: (i, 0)),
        out_shape=jax.ShapeDtypeStruct((N, H), jnp.float32),
    )(x_emb, scat, W1, W2, b)


def _node_dense_kernel(x_ref, vec_ref, lng_ref, lnb_ref, qW_ref, qb_ref,
                       kW_ref, kb_ref, vWx_ref, vbx_ref, vW1_ref, vb1_ref,
                       vW2_ref, vb2_ref, vecW_ref,
                       q_ref, kvv_ref, vecdot_ref, vec3_ref):
    x = x_ref[...]
    vec = vec_ref[...]
    xn = _ln_block(x, lng_ref[...], lnb_ref[...])
    q_ref[...] = jnp.dot(xn, qW_ref[...], preferred_element_type=jnp.float32) + qb_ref[...]
    kk = jnp.dot(xn, kW_ref[...], preferred_element_type=jnp.float32) + kb_ref[...]
    vx = jnp.dot(xn, vWx_ref[...], preferred_element_type=jnp.float32) + vbx_ref[...]
    v1 = jnp.dot(xn, vW1_ref[...], preferred_element_type=jnp.float32) + vb1_ref[...]
    v2 = jnp.dot(xn, vW2_ref[...], preferred_element_type=jnp.float32) + vb2_ref[...]
    ah = jnp.concatenate([kk, vx, v1, v2], axis=1)
    bh = jnp.concatenate([vec, jnp.zeros_like(kk)], axis=1)
    au = lax.bitcast_convert_type(ah, jnp.uint32)
    au = (au + jnp.uint32(0x8000)) >> 16
    bu = lax.bitcast_convert_type(bh, jnp.uint32)
    bu = (bu + jnp.uint32(0x8000)) & jnp.uint32(0xFFFF0000)
    kvv_ref[...] = lax.bitcast_convert_type(au | bu, jnp.int32)
    vecW = vecW_ref[...]
    vd = jnp.zeros_like(x)
    v3s = []
    for c in range(3):
        vp_c = jnp.dot(vec[:, c * H:(c + 1) * H], vecW,
                       preferred_element_type=jnp.float32)
        vd = vd + vp_c[:, 0:H] * vp_c[:, H:2 * H]
        v3s.append(vp_c[:, 2 * H:3 * H])
    vecdot_ref[...] = vd
    vec3_ref[...] = jnp.concatenate(v3s, axis=1)


def _node_dense(x, vec2d, lng, lnb, qW, qb, kW, kb, vWx, vbx, vW1, vb1,
                vW2, vb2, vecW):
    B = 1000
    full = lambda s: pl.BlockSpec(s, lambda i: tuple(0 for _ in s))
    out_shapes = (
        jax.ShapeDtypeStruct((N, H), jnp.float32),
        jax.ShapeDtypeStruct((N, 4 * H), jnp.int32),
        jax.ShapeDtypeStruct((N, H), jnp.float32),
        jax.ShapeDtypeStruct((N, 3 * H), jnp.float32),
    )
    return pl.pallas_call(
        _node_dense_kernel,
        grid=(N // B,),
        in_specs=[
            pl.BlockSpec((B, H), lambda i: (i, 0)),
            pl.BlockSpec((B, 3 * H), lambda i: (i, 0)),
            full((H,)), full((H,)),
            full((H, H)), full((H,)),
            full((H, H)), full((H,)),
            full((H, H)), full((H,)),
            full((H, H)), full((H,)),
            full((H, H)), full((H,)),
            full((H, 3 * H)),
        ],
        out_specs=[
            pl.BlockSpec((B, H), lambda i: (i, 0)),
            pl.BlockSpec((B, 4 * H), lambda i: (i, 0)),
            pl.BlockSpec((B, H), lambda i: (i, 0)),
            pl.BlockSpec((B, 3 * H), lambda i: (i, 0)),
        ],
        out_shape=out_shapes,
    )(x, vec2d, lng, lnb, qW, qb, kW, kb, vWx, vbx, vW1, vb1, vW2, vb2, vecW)


def _edge_layer_kernel(geo_ref, qd_ref, kvv_ref,
                       dkW_ref, dkb_ref, dWx_ref, dbx_ref, dW1_ref, db1_ref,
                       dW2_ref, db2_ref, selT_ref, selB_ref, out_ref):
    geo = geo_ref[...]
    attr = geo[:, 0:NRBF_P]
    cc = geo[:, NRBF_P + 1:NRBF_P + 2]
    d_ij = geo[:, NRBF_P + 2:NRBF_P + 5]
    w = lax.bitcast_convert_type(kvv_ref[...], jnp.uint32)
    kvv = lax.bitcast_convert_type(w << 16, jnp.float32)
    vecs = lax.bitcast_convert_type(w & jnp.uint32(0xFFFF0000),
                                    jnp.float32)
    k_s = kvv[:, 0:H]
    vx_s = kvv[:, H:2 * H]
    v1_s = kvv[:, 2 * H:3 * H]
    v2_s = kvv[:, 3 * H:4 * H]
    dk = _silu(jnp.dot(attr, dkW_ref[...], preferred_element_type=jnp.float32) + dkb_ref[...])
    dvx = _silu(jnp.dot(attr, dWx_ref[...], preferred_element_type=jnp.float32) + dbx_ref[...])
    dv1 = _silu(jnp.dot(attr, dW1_ref[...], preferred_element_type=jnp.float32) + db1_ref[...])
    dv2 = _silu(jnp.dot(attr, dW2_ref[...], preferred_element_type=jnp.float32) + db2_ref[...])
    prod = qd_ref[...].astype(jnp.float32) * k_s * dk
    attn = jnp.dot(prod, selT_ref[...], preferred_element_type=jnp.float32)
    attn = _silu(attn) * cc
    attnb = jnp.dot(attn, selB_ref[...], preferred_element_type=jnp.float32)
    xm = vx_s * dvx * attnb
    vm1 = v1_s * dv1 * attnb
    vm2 = v2_s * dv2 * attnb
    outs = [xm]
    for c in range(3):
        vec_c = vecs[:, c * H:(c + 1) * H]
        outs.append(vec_c * vm1 + vm2 * d_ij[:, c:c + 1])
    out_ref[...] = jnp.concatenate(outs, axis=1)


def _edge_layer(geo, q_dst, kvv_src, dkW, dkb, dWx, dbx, dW1, db1,
                dW2, db2, selT, selB):
    B = 1024
    full = lambda s: pl.BlockSpec(s, lambda i: tuple(0 for _ in s))
    return pl.pallas_call(
        _edge_layer_kernel,
        grid=(EP // B,),
        in_specs=[
            pl.BlockSpec((B, H), lambda i: (i, 0)),
            pl.BlockSpec((B, H), lambda i: (i, 0)),
            pl.BlockSpec((B, 4 * H), lambda i: (i, 0)),
            full((NRBF_P, H)), full((H,)),
            full((NRBF_P, H)), full((H,)),
            full((NRBF_P, H)), full((H,)),
            full((NRBF_P, H)), full((H,)),
            full((H, NH)), full((NH, H)),
        ],
        out_specs=pl.BlockSpec((B, 4 * H), lambda i: (i, 0)),
        out_shape=jax.ShapeDtypeStruct((EP, 4 * H), jnp.float32),
    )(geo, q_dst, kvv_src, dkW, dkb, dWx, dbx, dW1, db1, dW2, db2, selT, selB)


def _epi_kernel(x_ref, vec_ref, vd_ref, v3_ref, sc_ref, oW_ref, ob_ref,
                x_out, vec_out):
    x_agg = sc_ref[0, 0] + sc_ref[1, 0]
    vec_agg = jnp.concatenate(
        [sc_ref[0, 1 + c] + sc_ref[1, 1 + c] for c in range(3)], axis=1)
    o = jnp.dot(x_agg, oW_ref[...], preferred_element_type=jnp.float32) + ob_ref[...]
    o1 = o[:, 0:H]
    o2 = o[:, H:2 * H]
    o3 = o[:, 2 * H:3 * H]
    x_out[...] = x_ref[...] + vd_ref[...] * o2 + o3
    o1t = jnp.concatenate([o1, o1, o1], axis=1)
    vec_out[...] = vec_ref[...] + v3_ref[...] * o1t + vec_agg


def _epilogue(x, vec2d, vec_dot, vec3, scat, oW, ob):
    B = 1000
    full = lambda s: pl.BlockSpec(s, lambda i: tuple(0 for _ in s))
    return pl.pallas_call(
        _epi_kernel,
        grid=(N // B,),
        in_specs=[
            pl.BlockSpec((B, H), lambda i: (i, 0)),
            pl.BlockSpec((B, 3 * H), lambda i: (i, 0)),
            pl.BlockSpec((B, H), lambda i: (i, 0)),
            pl.BlockSpec((B, 3 * H), lambda i: (i, 0)),
            pl.BlockSpec((NC, 4, B, H), lambda i: (0, 0, i, 0)),
            full((H, 3 * H)), full((3 * H,)),
        ],
        out_specs=[
            pl.BlockSpec((B, H), lambda i: (i, 0)),
            pl.BlockSpec((B, 3 * H), lambda i: (i, 0)),
        ],
        out_shape=(
            jax.ShapeDtypeStruct((N, H), jnp.float32),
            jax.ShapeDtypeStruct((N, 3 * H), jnp.float32),
        ),
    )(x, vec2d, vec_dot, vec3, scat, oW, ob)


def _ln_kernel(x_ref, g_ref, b_ref, out_ref):
    out_ref[...] = _ln_block(x_ref[...], g_ref[...], b_ref[...])


def _final_ln(x, g, b):
    B = 1000
    full = lambda s: pl.BlockSpec(s, lambda i: tuple(0 for _ in s))
    return pl.pallas_call(
        _ln_kernel,
        grid=(N // B,),
        in_specs=[
            pl.BlockSpec((B, H), lambda i: (i, 0)),
            full((H,)), full((H,)),
        ],
        out_specs=pl.BlockSpec((B, H), lambda i: (i, 0)),
        out_shape=jax.ShapeDtypeStruct((N, H), jnp.float32),
    )(x, g, b)


def kernel(pos, elems, edge_index, emb_table, ne_emb_table, ne_dist_W,
           ne_dist_b, ne_comb_W, ne_comb_b, rbf_means, rbf_betas, ln_g, ln_b,
           qW, qb, kW, kb, vW, vb, oW, ob, vecW, dkW, dkb, dvW, dvb,
           outn_g, outn_b):
    f32 = jnp.float32
    src = edge_index[0].astype(jnp.int32)
    dst = edge_index[1].astype(jnp.int32)
    src_p = jnp.pad(src, (0, EP - E))
    dst_p = jnp.pad(dst, (0, EP - E))
    elems_p = jnp.pad(elems.astype(jnp.int32), (0, NP - N))

    ptab = jnp.concatenate(
        [pos.astype(f32), jnp.arange(N, dtype=f32)[:, None],
         jnp.zeros((N, H - 4), f32)], axis=1)
    means = jnp.pad(rbf_means, (0, NRBF_P - NRBF))
    betas = jnp.pad(rbf_betas, (0, NRBF_P - NRBF))
    ne_dist_Wp = jnp.pad(ne_dist_W, ((0, NRBF_P - NRBF), (0, 0)))

    selT = (jnp.arange(H)[:, None] // HD == jnp.arange(NH)[None, :]).astype(f32)
    selB = selT.T

    def _perm_v(W, b):
        Wr = W.reshape(-1, NH, 3, HD)
        br = b.reshape(NH, 3, HD)
        Ws = [Wr[:, :, s, :].reshape(-1, H) for s in range(3)]
        bs = [br[:, s, :].reshape(H) for s in range(3)]
        return Ws, bs

    zeros_acc = jnp.zeros((NACC, H), f32)

    embcat = jnp.concatenate([emb_table, ne_emb_table], axis=0)
    elems2 = jnp.concatenate([elems_p, elems_p + MAX_Z])
    emb2 = _sc_gather(embcat, elems2, H)
    x_emb = emb2[:N]
    ptab2 = jnp.concatenate([ptab, emb2[NP:NP + N]], axis=0)
    idx3 = jnp.concatenate([src_p, dst_p, src_p + N])
    pp = _sc_gather(ptab2, idx3, H)
    geo = _geo_edge(pp, means, betas)
    ne_msg = _ne_edge(geo, pp, ne_dist_Wp, ne_dist_b)
    ne_scat = _sc_scatter(ne_msg, dst_p, zeros_acc, 1)
    x = _ne_comb(x_emb, ne_scat, ne_comb_W[0:H], ne_comb_W[H:2 * H], ne_comb_b)

    vec2d = jnp.zeros((N, 3 * H), f32)
    for l in range(L):
        (vWs, vbs) = _perm_v(vW[l], vb[l])
        (dWs, dbs) = _perm_v(jnp.pad(dvW[l], ((0, NRBF_P - NRBF), (0, 0))),
                             dvb[l])
        dkWp = jnp.pad(dkW[l], ((0, NRBF_P - NRBF), (0, 0)))
        q, kvv, vec_dot, vec3 = _node_dense(
            x, vec2d, ln_g[l], ln_b[l], qW[l], qb[l], kW[l], kb[l],
            vWs[0], vbs[0], vWs[1], vbs[1], vWs[2], vbs[2], vecW[l])
        q_dst = _sc_gather(q, dst_p, H)
        kvv_src = _sc_gather(kvv, src_p, 4 * H)
        msgs = _edge_layer(geo, q_dst, kvv_src,
                           dkWp, dkb[l], dWs[0], dbs[0], dWs[1], dbs[1],
                           dWs[2], dbs[2], selT, selB)
        scat = _sc_scatter(msgs, dst_p, zeros_acc, 4)
        x, vec2d = _epilogue(x, vec2d, vec_dot, vec3, scat, oW[l], ob[l])

    x = _final_ln(x, outn_g, outn_b)
    return x, vec2d.reshape(N, 3, H)
